# baseline (speedup 1.0000x reference)
import functools

import jax
import jax.numpy as jnp
from jax import lax
from jax.experimental import pallas as pl
from jax.experimental.pallas import tpu as pltpu
from jax.experimental.pallas import tpu_sc as plsc

NC = 2
NS = 16
NW = NC * NS
K = 1024
L = 16
TK = K // 128
SUBS = (2048, 1024, 512, 256, 128, 64, 32, 16)


@functools.lru_cache(maxsize=None)
def _build(B, TOTAL, NSEL):
    C = (NSEL + K - 1) // K
    NSEL_P = C * K
    G = B // 8
    TASKS = G * C
    ITERS = (TASKS + NW - 1) // NW
    HALF = TOTAL // 2
    WIN = HALF - 56
    CT = (NSEL + 127) // 128
    LAST_TK = CT - (C - 1) * TK
    mesh = plsc.VectorSubcoreMesh(core_axis_name="c", subcore_axis_name="s")

    @functools.partial(
        pl.kernel,
        out_type=jax.ShapeDtypeStruct((G, CT, 8, 128), jnp.float32),
        mesh=mesh,
        scratch_types=[
            pltpu.VMEM((K,), jnp.int32),
            pltpu.VMEM((K,), jnp.int32),
            pltpu.VMEM((K,), jnp.int32),
            pltpu.VMEM((K,), jnp.int32),
            pltpu.VMEM((K + L,), jnp.int32),
            pltpu.VMEM((K, L), jnp.float32),
            pltpu.VMEM((K, L), jnp.float32),
            pltpu.VMEM((TK, 8, 128), jnp.float32),
            pltpu.SemaphoreType.DMA,
            pltpu.SemaphoreType.DMA,
        ],
        compiler_params=pltpu.CompilerParams(
            needs_layout_passes=False, use_tc_tiling_on_sc=False
        ),
    )
    def gather_kernel(tab_hbm, idx_hbm, out_hbm, idx_v, row_v, lane_v,
                      pos_v, uniq_v, rows_a, rows_b, stage_v, sem_a, sem_b):
        wid = lax.axis_index("s") * NC + lax.axis_index("c")
        bufs = (rows_a, rows_b)
        sems = (sem_a, sem_b)
        iota = lax.iota(jnp.int32, L)

        @pl.loop(0, ITERS)
        def _task(i):
            t = wid + i * NW

            @pl.when(t < TASKS)
            def _run():
                g = t % G
                c = t // G
                start = c * K
                pltpu.sync_copy(idx_hbm.at[pl.ds(start, K)], idx_v)

                @pl.loop(0, K // L)
                def _rows(j):
                    iv = idx_v[pl.ds(j * L, L)]
                    row_v[pl.ds(j * L, L)] = (
                        lax.shift_left(lax.shift_right_logical(iv, 7), 6)
                        | (lax.shift_right_logical(iv, 4) & 7)
                    )
                    lane_v[pl.ds(j * L, L)] = iv & (L - 1)

                @pl.loop(0, K // L, init_carry=jnp.int32(0))
                def _dedup(j, carry):
                    rv16 = row_v[pl.ds(j * L, L)]
                    prev16 = plsc.load_gather(
                        row_v, [lax.max(j * L - 1 + iota, jnp.int32(0))]
                    )
                    m = (rv16 != prev16) | ((j * L + iota) == 0)
                    cs = plsc.cumsum(m.astype(jnp.int32))
                    pos_v[pl.ds(j * L, L)] = carry + cs - 1
                    plsc.store_compressed(
                        uniq_v.at[pl.ds(carry, L)], rv16, mask=m
                    )
                    return carry + lax.reduce_max(cs, (0,))

                u_cnt = _dedup
                last16 = plsc.load_gather(
                    uniq_v, [u_cnt - 1 + 0 * iota]
                )
                uniq_v[pl.ds(u_cnt, L)] = last16
                u_ceil = (u_cnt + L - 1) & jnp.int32(-L)

                offs = []
                boff = jnp.int32(0)
                for sz in SUBS:
                    offs.append(boff)
                    boff = boff + lax.select(
                        (u_ceil & sz) != 0, jnp.int32(sz), jnp.int32(0)
                    )

                def start_row(r, buf, sem):
                    win = tab_hbm.at[pl.ds(g * HALF + r * 8, WIN)]
                    for sz, off in zip(SUBS, offs):
                        @pl.when((u_ceil & sz) != 0)
                        def _s(sz=sz, off=off):
                            pltpu.async_copy(
                                win.at[uniq_v.at[pl.ds(off, sz)]],
                                buf.at[pl.ds(off, sz)],
                                sem,
                            )

                def wait_row(r, buf, sem):
                    win = tab_hbm.at[pl.ds(g * HALF + r * 8, WIN)]
                    for sz, off in zip(SUBS, offs):
                        @pl.when((u_ceil & sz) != 0)
                        def _w(sz=sz, off=off):
                            pltpu.make_async_copy(
                                win.at[uniq_v.at[pl.ds(off, sz)]],
                                buf.at[pl.ds(off, sz)],
                                sem,
                            ).wait()

                start_row(0, bufs[0], sems[0])
                for r in range(8):
                    if r < 7:
                        start_row(r + 1, bufs[(r + 1) % 2], sems[(r + 1) % 2])
                    wait_row(r, bufs[r % 2], sems[r % 2])
                    buf = bufs[r % 2]

                    @pl.loop(0, K // L)
                    def _extract(j):
                        pos16 = pos_v[pl.ds(j * L, L)]
                        lane16 = lane_v[pl.ds(j * L, L)]
                        v16 = plsc.load_gather(buf, [pos16, lane16])
                        stage_v[j // 8, r, pl.ds((j % 8) * L, L)] = v16

                @pl.when(c != C - 1)
                def _store():
                    pltpu.sync_copy(
                        stage_v,
                        out_hbm.at[g, pl.ds(c * TK, TK)],
                    )

                @pl.when(c == C - 1)
                def _store_tail():
                    pltpu.sync_copy(
                        stage_v.at[pl.ds(0, LAST_TK)],
                        out_hbm.at[g, pl.ds((C - 1) * TK, LAST_TK)],
                    )

    return gather_kernel, NSEL_P, CT


def kernel(params, rp_cat, single_grad, flat_cat):
    B, TOTAL = params.shape
    NSEL = rp_cat.shape[0]
    if TOTAL == NSEL:
        return params
    gk, NSEL_P, CT = _build(B, TOTAL, NSEL)
    rp = rp_cat.astype(jnp.int32)
    if NSEL_P != NSEL:
        rp = jnp.pad(rp, (0, NSEL_P - NSEL), mode="edge")
    tab = (
        params.reshape(B // 8, 8, TOTAL // 128, 128)
        .transpose(0, 2, 1, 3)
        .reshape(B * TOTAL // L, L)
    )
    out4 = gk(tab, rp)
    out = (
        out4.transpose(0, 2, 1, 3)
        .reshape(B, CT * 128)[:, :NSEL]
    )
    return out


# speedup vs baseline: 1.1221x; 1.1221x over previous
import functools

import jax
import jax.numpy as jnp
from jax import lax
from jax.experimental import pallas as pl
from jax.experimental.pallas import tpu as pltpu
from jax.experimental.pallas import tpu_sc as plsc

NC = 2
NS = 16
NW = NC * NS
K = 2048
L = 16
TK = K // 128
SUBS = (2048, 1024, 512, 256, 128, 64, 32, 16)


@functools.lru_cache(maxsize=None)
def _build(B, TOTAL, NSEL):
    C = (NSEL + K - 1) // K
    NSEL_P = C * K
    G = B // 8
    TASKS = G * C
    ITERS = (TASKS + NW - 1) // NW
    HALF = TOTAL // 2
    WIN = HALF - 56
    CT = (NSEL + 127) // 128
    LAST_TK = CT - (C - 1) * TK
    mesh = plsc.VectorSubcoreMesh(core_axis_name="c", subcore_axis_name="s")

    @functools.partial(
        pl.kernel,
        out_type=jax.ShapeDtypeStruct((G, CT, 8, 128), jnp.float32),
        mesh=mesh,
        scratch_types=[
            pltpu.VMEM((K,), jnp.int32),
            pltpu.VMEM((K,), jnp.int32),
            pltpu.VMEM((K,), jnp.int32),
            pltpu.VMEM((K,), jnp.int32),
            pltpu.VMEM((K,), jnp.int32),
            pltpu.VMEM((K,), jnp.int32),
            pltpu.VMEM((K + L,), jnp.int32),
            pltpu.VMEM((K + L,), jnp.int32),
            pltpu.VMEM((K, L), jnp.float32),
            pltpu.VMEM((K, L), jnp.float32),
            pltpu.VMEM((TK, 8, 128), jnp.float32),
            pltpu.SemaphoreType.DMA,
            pltpu.SemaphoreType.DMA,
        ],
        compiler_params=pltpu.CompilerParams(
            needs_layout_passes=False, use_tc_tiling_on_sc=False
        ),
    )
    def gather_kernel(tab_hbm, idx_hbm, out_hbm, idx_v, row_v, lane_a,
                      lane_b, pos_a, pos_b, uniq_a, uniq_b, rows_a, rows_b,
                      stage_v, sem_a, sem_b):
        wid = lax.axis_index("s") * NC + lax.axis_index("c")
        bufs = (rows_a, rows_b)
        sems = (sem_a, sem_b)
        lanes = (lane_a, lane_b)
        poss = (pos_a, pos_b)
        uniqs = (uniq_a, uniq_b)
        iota = lax.iota(jnp.int32, L)

        def prologue(t, x):
            lane_v, pos_v, uniq_v = lanes[x], poss[x], uniqs[x]
            c = t // G
            pltpu.sync_copy(idx_hbm.at[pl.ds(c * K, K)], idx_v)

            @pl.loop(0, K // L)
            def _rows(j):
                iv = idx_v[pl.ds(j * L, L)]
                row_v[pl.ds(j * L, L)] = (
                    lax.shift_left(lax.shift_right_logical(iv, 7), 6)
                    | (lax.shift_right_logical(iv, 4) & 7)
                )
                lane_v[pl.ds(j * L, L)] = iv & (L - 1)

            @pl.loop(0, K // L, init_carry=jnp.int32(0))
            def _dedup(j, carry):
                rv16 = row_v[pl.ds(j * L, L)]
                prev16 = plsc.load_gather(
                    row_v, [lax.max(j * L - 1 + iota, jnp.int32(0))]
                )
                m = (rv16 != prev16) | ((j * L + iota) == 0)
                cs = plsc.cumsum(m.astype(jnp.int32))
                pos_v[pl.ds(j * L, L)] = carry + cs - 1
                plsc.store_compressed(
                    uniq_v.at[pl.ds(carry, L)], rv16, mask=m
                )
                return carry + lax.reduce_max(cs, (0,))

            u_cnt = _dedup
            last16 = plsc.load_gather(uniq_v, [u_cnt - 1 + 0 * iota])
            uniq_v[pl.ds(u_cnt, L)] = last16
            return (u_cnt + L - 1) & jnp.int32(-L)

        def sub_offsets(u_ceil):
            offs, boff = [], jnp.int32(0)
            for sz in SUBS:
                offs.append(boff)
                boff = boff + lax.select(
                    (u_ceil & sz) != 0, jnp.int32(sz), jnp.int32(0)
                )
            return offs

        def row_copies(t, x, u_ceil, offs, r, buf, sem):
            g = t % G
            win = tab_hbm.at[pl.ds(g * HALF + r * 8, WIN)]
            for sz, off in zip(SUBS, offs):
                yield (u_ceil & sz) != 0, pltpu.make_async_copy(
                    win.at[uniqs[x].at[pl.ds(off, sz)]],
                    buf.at[pl.ds(off, sz)],
                    sem,
                )

        def issue_row(t, x, u_ceil, offs, r):
            for cond, cp in row_copies(t, x, u_ceil, offs, r,
                                       bufs[r % 2], sems[r % 2]):
                @pl.when(cond)
                def _s(cp=cp):
                    cp.start()

        def part1(t, x, u_ceil):
            offs = sub_offsets(u_ceil)
            issue_row(t, x, u_ceil, offs, 0)
            issue_row(t, x, u_ceil, offs, 1)

        def part2(t, x, u_ceil):
            g = t % G
            c = t // G
            offs = sub_offsets(u_ceil)
            lane_v, pos_v = lanes[x], poss[x]
            for r in range(8):
                for cond, cp in row_copies(t, x, u_ceil, offs, r,
                                           bufs[r % 2], sems[r % 2]):
                    @pl.when(cond)
                    def _w(cp=cp):
                        cp.wait()
                buf = bufs[r % 2]

                @pl.loop(0, K // L)
                def _extract(j, r=r, buf=buf):
                    pos16 = pos_v[pl.ds(j * L, L)]
                    lane16 = lane_v[pl.ds(j * L, L)]
                    v16 = plsc.load_gather(buf, [pos16, lane16])
                    stage_v[j // 8, r, pl.ds((j % 8) * L, L)] = v16

                if r < 6:
                    issue_row(t, x, u_ceil, offs, r + 2)

            @pl.when(c != C - 1)
            def _store():
                pltpu.sync_copy(stage_v, out_hbm.at[g, pl.ds(c * TK, TK)])

            @pl.when(c == C - 1)
            def _store_tail():
                pltpu.sync_copy(
                    stage_v.at[pl.ds(0, LAST_TK)],
                    out_hbm.at[g, pl.ds((C - 1) * TK, LAST_TK)],
                )

        uc0 = prologue(wid, 0)

        @pl.loop(0, (ITERS + 1) // 2, init_carry=uc0)
        def _pair(m, uc_a):
            t0 = wid + 2 * m * NW
            t1 = t0 + NW
            t2 = t0 + 2 * NW

            @pl.when(t0 < TASKS)
            def _a1():
                part1(t0, 0, uc_a)

            uc_b = prologue(lax.min(t1, jnp.int32(TASKS - 1)), 1)

            @pl.when(t0 < TASKS)
            def _a2():
                part2(t0, 0, uc_a)

            @pl.when(t1 < TASKS)
            def _b1():
                part1(t1, 1, uc_b)

            uc_a2 = prologue(lax.min(t2, jnp.int32(TASKS - 1)), 0)

            @pl.when(t1 < TASKS)
            def _b2():
                part2(t1, 1, uc_b)

            return uc_a2

    return gather_kernel, NSEL_P, CT


def kernel(params, rp_cat, single_grad, flat_cat):
    B, TOTAL = params.shape
    NSEL = rp_cat.shape[0]
    if TOTAL == NSEL:
        return params
    gk, NSEL_P, CT = _build(B, TOTAL, NSEL)
    rp = rp_cat.astype(jnp.int32)
    if NSEL_P != NSEL:
        rp = jnp.pad(rp, (0, NSEL_P - NSEL), mode="edge")
    tab = (
        params.reshape(B // 8, 8, TOTAL // 128, 128)
        .transpose(0, 2, 1, 3)
        .reshape(B * TOTAL // L, L)
    )
    out4 = gk(tab, rp)
    out = (
        out4.transpose(0, 2, 1, 3)
        .reshape(B, CT * 128)[:, :NSEL]
    )
    return out


# speedup vs baseline: 1.1896x; 1.0602x over previous
import functools

import jax
import jax.numpy as jnp
from jax import lax
from jax.experimental import pallas as pl
from jax.experimental.pallas import tpu as pltpu
from jax.experimental.pallas import tpu_sc as plsc

NC = 2
NS = 16
NW = NC * NS
K = 1664
L = 16
TK = K // 128
SUBS = (2048, 1024, 512, 256, 128, 64, 32, 16)


@functools.lru_cache(maxsize=None)
def _build(B, TOTAL, NSEL):
    C = (NSEL + K - 1) // K
    NSEL_P = C * K
    G = B // 8
    TASKS = G * C
    ITERS = (TASKS + NW - 1) // NW
    HALF = TOTAL // 2
    WIN = HALF - 56
    CT = (NSEL + 127) // 128
    LAST_TK = CT - (C - 1) * TK
    mesh = plsc.VectorSubcoreMesh(core_axis_name="c", subcore_axis_name="s")

    @functools.partial(
        pl.kernel,
        out_type=jax.ShapeDtypeStruct((G, CT, 8, 128), jnp.float32),
        mesh=mesh,
        scratch_types=[
            pltpu.VMEM((K,), jnp.int32),
            pltpu.VMEM((K,), jnp.int32),
            pltpu.VMEM((K,), jnp.int32),
            pltpu.VMEM((K,), jnp.int32),
            pltpu.VMEM((K,), jnp.int32),
            pltpu.VMEM((K,), jnp.int32),
            pltpu.VMEM((K + L,), jnp.int32),
            pltpu.VMEM((K + L,), jnp.int32),
            pltpu.VMEM((K, L), jnp.float32),
            pltpu.VMEM((K, L), jnp.float32),
            pltpu.VMEM((TK, 8, 128), jnp.float32),
            pltpu.SemaphoreType.DMA,
            pltpu.SemaphoreType.DMA,
        ],
        compiler_params=pltpu.CompilerParams(
            needs_layout_passes=False, use_tc_tiling_on_sc=False
        ),
    )
    def gather_kernel(tab_hbm, idx_hbm, out_hbm, idx_v, row_v, lane_a,
                      lane_b, pos_a, pos_b, uniq_a, uniq_b, rows_a, rows_b,
                      stage_v, sem_a, sem_b):
        wid = lax.axis_index("s") * NC + lax.axis_index("c")
        bufs = (rows_a, rows_b)
        sems = (sem_a, sem_b)
        lanes = (lane_a, lane_b)
        poss = (pos_a, pos_b)
        uniqs = (uniq_a, uniq_b)
        iota = lax.iota(jnp.int32, L)

        def prologue(t, x):
            lane_v, pos_v, uniq_v = lanes[x], poss[x], uniqs[x]
            c = t // G
            pltpu.sync_copy(idx_hbm.at[pl.ds(c * K, K)], idx_v)

            @pl.loop(0, K // L)
            def _rows(j):
                iv = idx_v[pl.ds(j * L, L)]
                row_v[pl.ds(j * L, L)] = (
                    lax.shift_left(lax.shift_right_logical(iv, 7), 6)
                    | (lax.shift_right_logical(iv, 4) & 7)
                )
                lane_v[pl.ds(j * L, L)] = iv & (L - 1)

            @pl.loop(0, K // L, init_carry=jnp.int32(0))
            def _dedup(j, carry):
                rv16 = row_v[pl.ds(j * L, L)]
                prev16 = plsc.load_gather(
                    row_v, [lax.max(j * L - 1 + iota, jnp.int32(0))]
                )
                m = (rv16 != prev16) | ((j * L + iota) == 0)
                cs = plsc.cumsum(m.astype(jnp.int32))
                pos_v[pl.ds(j * L, L)] = carry + cs - 1
                plsc.store_compressed(
                    uniq_v.at[pl.ds(carry, L)], rv16, mask=m
                )
                return carry + lax.reduce_max(cs, (0,))

            u_cnt = _dedup
            last16 = plsc.load_gather(uniq_v, [u_cnt - 1 + 0 * iota])
            uniq_v[pl.ds(u_cnt, L)] = last16
            return (u_cnt + L - 1) & jnp.int32(-L)

        def sub_offsets(u_ceil):
            offs, boff = [], jnp.int32(0)
            for sz in SUBS:
                offs.append(boff)
                boff = boff + lax.select(
                    (u_ceil & sz) != 0, jnp.int32(sz), jnp.int32(0)
                )
            return offs

        def row_copies(t, x, u_ceil, offs, r, buf, sem):
            g = t % G
            win = tab_hbm.at[pl.ds(g * HALF + r * 8, WIN)]
            for sz, off in zip(SUBS, offs):
                yield (u_ceil & sz) != 0, pltpu.make_async_copy(
                    win.at[uniqs[x].at[pl.ds(off, sz)]],
                    buf.at[pl.ds(off, sz)],
                    sem,
                )

        def issue_row(t, x, u_ceil, offs, r):
            for cond, cp in row_copies(t, x, u_ceil, offs, r,
                                       bufs[r % 2], sems[r % 2]):
                @pl.when(cond)
                def _s(cp=cp):
                    cp.start()

        def part1(t, x, u_ceil):
            offs = sub_offsets(u_ceil)
            issue_row(t, x, u_ceil, offs, 0)
            issue_row(t, x, u_ceil, offs, 1)

        def part2(t, x, u_ceil):
            g = t % G
            c = t // G
            offs = sub_offsets(u_ceil)
            lane_v, pos_v = lanes[x], poss[x]
            for r in range(8):
                for cond, cp in row_copies(t, x, u_ceil, offs, r,
                                           bufs[r % 2], sems[r % 2]):
                    @pl.when(cond)
                    def _w(cp=cp):
                        cp.wait()
                buf = bufs[r % 2]

                @pl.loop(0, K // L)
                def _extract(j, r=r, buf=buf):
                    pos16 = pos_v[pl.ds(j * L, L)]
                    lane16 = lane_v[pl.ds(j * L, L)]
                    v16 = plsc.load_gather(buf, [pos16, lane16])
                    stage_v[j // 8, r, pl.ds((j % 8) * L, L)] = v16

                if r < 6:
                    issue_row(t, x, u_ceil, offs, r + 2)

            @pl.when(c != C - 1)
            def _store():
                pltpu.sync_copy(stage_v, out_hbm.at[g, pl.ds(c * TK, TK)])

            @pl.when(c == C - 1)
            def _store_tail():
                pltpu.sync_copy(
                    stage_v.at[pl.ds(0, LAST_TK)],
                    out_hbm.at[g, pl.ds((C - 1) * TK, LAST_TK)],
                )

        uc0 = prologue(wid, 0)

        @pl.loop(0, (ITERS + 1) // 2, init_carry=uc0)
        def _pair(m, uc_a):
            t0 = wid + 2 * m * NW
            t1 = t0 + NW
            t2 = t0 + 2 * NW

            @pl.when(t0 < TASKS)
            def _a1():
                part1(t0, 0, uc_a)

            uc_b = prologue(lax.min(t1, jnp.int32(TASKS - 1)), 1)

            @pl.when(t0 < TASKS)
            def _a2():
                part2(t0, 0, uc_a)

            @pl.when(t1 < TASKS)
            def _b1():
                part1(t1, 1, uc_b)

            uc_a2 = prologue(lax.min(t2, jnp.int32(TASKS - 1)), 0)

            @pl.when(t1 < TASKS)
            def _b2():
                part2(t1, 1, uc_b)

            return uc_a2

    return gather_kernel, NSEL_P, CT


def kernel(params, rp_cat, single_grad, flat_cat):
    B, TOTAL = params.shape
    NSEL = rp_cat.shape[0]
    if TOTAL == NSEL:
        return params
    gk, NSEL_P, CT = _build(B, TOTAL, NSEL)
    rp = rp_cat.astype(jnp.int32)
    if NSEL_P != NSEL:
        rp = jnp.pad(rp, (0, NSEL_P - NSEL), mode="edge")
    tab = (
        params.reshape(B // 8, 8, TOTAL // 128, 128)
        .transpose(0, 2, 1, 3)
        .reshape(B * TOTAL // L, L)
    )
    out4 = gk(tab, rp)
    out = (
        out4.transpose(0, 2, 1, 3)
        .reshape(B, CT * 128)[:, :NSEL]
    )
    return out
